# baseline (speedup 1.0000x reference)
import functools
import math

import jax
import jax.numpy as jnp
import numpy as np
from jax.experimental import pallas as pl
from jax.experimental.pallas import tpu as pltpu
from jax.sharding import Mesh, PartitionSpec

_INV_SQRT2 = 1.0 / math.sqrt(2.0)


def _block_kernel(x_ref, wtap_ref, dwb_ref, w1_ref, b1_ref, w2_ref, b2_ref,
                  o_ref, xpad_ref, xsh_ref, *, H, W, C, K, eps):
    P = K // 2
    M = H * W
    x = x_ref[...].astype(jnp.float32)
    xpad_ref[...] = jnp.zeros_like(xpad_ref)
    xpad_ref[P:P + H, P:P + W, :] = x
    for kx in range(K):
        xsh_ref[kx, :, :, :] = xpad_ref[:, kx:kx + W, :]
    acc = jnp.broadcast_to(dwb_ref[...].reshape(1, 1, C), (H, W, C))
    for ky in range(K):
        for kx in range(K):
            tap = wtap_ref[ky * K + kx, :].reshape(1, 1, C)
            acc = acc + xsh_ref[kx, ky:ky + H, :, :] * tap
    dw = acc.reshape(M, C)
    mean = jnp.mean(dw, axis=-1, keepdims=True)
    mean_sq = jnp.mean(dw * dw, axis=-1, keepdims=True)
    var = mean_sq - mean * mean
    y = (dw - mean) * jax.lax.rsqrt(var + eps)
    h = jnp.dot(y, w1_ref[...], preferred_element_type=jnp.float32)
    h = h + b1_ref[...]
    g = h * (1.0 + jax.lax.erf(h * _INV_SQRT2))
    z = jnp.dot(g, w2_ref[...], preferred_element_type=jnp.float32)
    z = z + b2_ref[...]
    out = x.reshape(M, C) + z
    o_ref[...] = out.reshape(H, W, C).astype(o_ref.dtype)


def _run_block(x, wtap, dwb, w1p, b1p, w2p, b2p, *, C, H, W, K, eps):
    NL = x.shape[0]
    P = K // 2
    H4 = w1p.shape[1]
    KK = wtap.shape[0]
    x_nhwc = jnp.transpose(x, (0, 2, 3, 1))

    def fullspec(shape):
        return pl.BlockSpec(shape, lambda n: (0,) * len(shape))

    y_nhwc = pl.pallas_call(
        functools.partial(_block_kernel, H=H, W=W, C=C, K=K, eps=eps),
        out_shape=jax.ShapeDtypeStruct((NL, H, W, C), x.dtype),
        grid=(NL,),
        in_specs=[
            pl.BlockSpec((None, H, W, C), lambda n: (n, 0, 0, 0)),
            fullspec((KK, C)),
            fullspec((1, C)),
            fullspec((C, H4)),
            fullspec((1, H4)),
            fullspec((H4, C)),
            fullspec((1, C)),
        ],
        out_specs=pl.BlockSpec((None, H, W, C), lambda n: (n, 0, 0, 0)),
        scratch_shapes=[
            pltpu.VMEM((H + 2 * P, W + 2 * P, C), jnp.float32),
            pltpu.VMEM((K, H + 2 * P, W, C), jnp.float32),
        ],
        compiler_params=pltpu.CompilerParams(
            dimension_semantics=("parallel",),
            vmem_limit_bytes=48 * 1024 * 1024),
    )(x_nhwc, wtap, dwb, w1p, b1p, w2p, b2p)
    return jnp.transpose(y_nhwc, (0, 3, 1, 2))


def kernel(x, dw_w, dw_b, ln_w, ln_b, w1, b1, w2, b2, gamma):
    N, C, H, W = x.shape
    K = 7
    H4 = w1.shape[0]
    eps = 1e-6

    wtap = dw_w.reshape(C, K * K).T.astype(jnp.float32)
    KK = ((K * K + 7) // 8) * 8
    wtap = jnp.pad(wtap, ((0, KK - K * K), (0, 0)))
    f32 = jnp.float32
    w1p = ln_w.astype(f32)[:, None] * w1.T.astype(f32)
    b1p = b1.astype(f32) + ln_b.astype(f32) @ w1.T.astype(f32)
    w2p = 0.5 * (w2.T.astype(f32) * gamma.astype(f32)[None, :])
    b2p = b2.astype(f32) * gamma.astype(f32)
    args = (wtap, dw_b.reshape(1, C).astype(f32), w1p,
            b1p.reshape(1, H4), w2p, b2p.reshape(1, C))

    run = functools.partial(_run_block, C=C, H=H, W=W, K=K, eps=eps)

    devs = jax.devices()
    nd = len(devs)
    while nd > 1 and N % nd != 0:
        nd -= 1
    if nd > 1:
        mesh = Mesh(np.array(devs[:nd]), ("b",))
        pb = PartitionSpec("b")
        pr = PartitionSpec()
        run = jax.shard_map(
            run, mesh=mesh,
            in_specs=(pb,) + (pr,) * len(args),
            out_specs=pb, check_vma=False)
    return run(x, *args)


# speedup vs baseline: 4.3914x; 4.3914x over previous
import functools
import math

import jax
import jax.numpy as jnp
from jax.experimental import pallas as pl
from jax.experimental.pallas import tpu as pltpu

_INV_SQRT2 = 1.0 / math.sqrt(2.0)


def _block_kernel(x_ref, wtap_ref, dwb_ref, w1_ref, b1_ref, w2_ref, b2_ref,
                  o_ref, xpad_ref, xsh_ref, *, H, W, C, K, eps):
    P = K // 2
    M = H * W
    Wp = W + 2 * P
    x = x_ref[...].astype(jnp.float32)
    xpad_ref[0:P, :, :] = jnp.zeros((P, Wp, C), jnp.float32)
    xpad_ref[P + H:, :, :] = jnp.zeros((P, Wp, C), jnp.float32)
    xpad_ref[P:P + H, 0:P, :] = jnp.zeros((H, P, C), jnp.float32)
    xpad_ref[P:P + H, P + W:, :] = jnp.zeros((H, P, C), jnp.float32)
    xpad_ref[P:P + H, P:P + W, :] = x
    for kx in range(K):
        xsh_ref[kx, :, :, :] = xpad_ref[:, kx:kx + W, :]
    acc = jnp.broadcast_to(dwb_ref[...].reshape(1, 1, C), (H, W, C))
    for ky in range(K):
        for kx in range(K):
            tap = wtap_ref[ky * K + kx, :].reshape(1, 1, C)
            acc = acc + xsh_ref[kx, ky:ky + H, :, :] * tap
    dw = acc.reshape(M, C)
    mean = jnp.mean(dw, axis=-1, keepdims=True)
    mean_sq = jnp.mean(dw * dw, axis=-1, keepdims=True)
    var = mean_sq - mean * mean
    y = (dw - mean) * jax.lax.rsqrt(var + eps)
    h = jnp.dot(y, w1_ref[...],
                preferred_element_type=jnp.float32) + b1_ref[...]
    g = h * (1.0 + jax.lax.erf(h))
    z = jnp.dot(g, w2_ref[...],
                preferred_element_type=jnp.float32) + b2_ref[...]
    out = x.reshape(M, C) + z
    o_ref[...] = out.reshape(H, W, C).astype(o_ref.dtype)


def _run_block(x, wtap, dwb, w1p, b1p, w2p, b2p, *, C, H, W, K, eps):
    NL = x.shape[0]
    P = K // 2
    H4 = w1p.shape[1]
    KK = wtap.shape[0]
    x_nhwc = jnp.transpose(x, (0, 2, 3, 1))

    def fullspec(shape):
        return pl.BlockSpec(shape, lambda n: (0,) * len(shape))

    y_nhwc = pl.pallas_call(
        functools.partial(_block_kernel, H=H, W=W, C=C, K=K, eps=eps),
        out_shape=jax.ShapeDtypeStruct((NL, H, W, C), x.dtype),
        grid=(NL,),
        in_specs=[
            pl.BlockSpec((None, H, W, C), lambda n: (n, 0, 0, 0)),
            fullspec((KK, C)),
            fullspec((1, C)),
            fullspec((C, H4)),
            fullspec((1, H4)),
            fullspec((H4, C)),
            fullspec((1, C)),
        ],
        out_specs=pl.BlockSpec((None, H, W, C), lambda n: (n, 0, 0, 0)),
        scratch_shapes=[
            pltpu.VMEM((H + 2 * P, W + 2 * P, C), jnp.float32),
            pltpu.VMEM((K, H + 2 * P, W, C), jnp.float32),
        ],
        compiler_params=pltpu.CompilerParams(
            dimension_semantics=("parallel",),
            vmem_limit_bytes=48 * 1024 * 1024),
    )(x_nhwc, wtap, dwb, w1p, b1p, w2p, b2p)
    return jnp.transpose(y_nhwc, (0, 3, 1, 2))


def kernel(x, dw_w, dw_b, ln_w, ln_b, w1, b1, w2, b2, gamma):
    N, C, H, W = x.shape
    K = 7
    H4 = w1.shape[0]
    eps = 1e-6

    wtap = dw_w.reshape(C, K * K).T.astype(jnp.float32)
    KK = ((K * K + 7) // 8) * 8
    wtap = jnp.pad(wtap, ((0, KK - K * K), (0, 0)))
    f32 = jnp.float32
    s = _INV_SQRT2
    w1p = s * ln_w.astype(f32)[:, None] * w1.T.astype(f32)
    b1p = s * (b1.astype(f32) + ln_b.astype(f32) @ w1.T.astype(f32))
    w2p = (0.5 / s) * (w2.T.astype(f32) * gamma.astype(f32)[None, :])
    b2p = b2.astype(f32) * gamma.astype(f32)
    args = (wtap, dw_b.reshape(1, C).astype(f32), w1p,
            b1p.reshape(1, H4), w2p, b2p.reshape(1, C))

    run = functools.partial(_run_block, C=C, H=H, W=W, K=K, eps=eps)
    return run(x, *args)
